# baseline (speedup 1.0000x reference)
import jax
import jax.numpy as jnp
from jax import lax
from jax.experimental import pallas as pl
from jax.experimental.pallas import tpu as pltpu
from jax.experimental.pallas import tpu_sc as plsc

N = 10000
NP = 10240
E = 320000
D_IN = 128
H = 64
FC_H = 32
NCLS = 32
NG = 128

NW = 32
CHUNK = 128
CPW = 79
EP = NW * CPW * CHUNK
CPW0 = 64
CPW1 = 2 * CPW - CPW0
RPT = NP // 16
WB = RPT // CHUNK

_f32 = jnp.float32


def _sc_mesh():
    return plsc.VectorSubcoreMesh(core_axis_name="c", subcore_axis_name="s")


_SC_PARAMS = pltpu.CompilerParams(use_tc_tiling_on_sc=False)


def _fill(buf, rows, cols, value):
    vals = jnp.full((16,), value, _f32)
    nc = cols // 16

    def st(k, c):
        buf[k // nc, pl.ds((k % nc) * 16, 16)] = vals
        return c

    lax.fori_loop(0, rows * nc, st, 0)


def _deg_body(dstw, out, accd, dstv, buf):
    cid = lax.axis_index("c")
    sid = lax.axis_index("s")
    wid = sid * 2 + cid
    _fill(buf, CHUNK, 16, 0.0)

    def z(k, c):
        rr = sid * RPT + k * CHUNK
        pltpu.sync_copy(buf, accd.at[pl.ds(rr, CHUNK)])
        return c

    lax.fori_loop(0, WB, z, 0)
    _fill(buf, CHUNK, 16, 1.0)
    pltpu.sync_copy(dstw.at[wid], dstv)
    ncha = jnp.where(cid == 0, CPW0, CPW1)
    plsc.subcore_barrier()

    def chunk(j, c):
        pltpu.sync_copy(buf, accd.at[dstv.at[j]], add=True)
        return c

    lax.fori_loop(0, ncha, chunk, 0)
    plsc.subcore_barrier()

    def wb(k, c):
        rr = sid * RPT + k * CHUNK
        pltpu.sync_copy(accd.at[pl.ds(rr, CHUNK)], out.at[cid, pl.ds(rr, CHUNK)])
        return c

    lax.fori_loop(0, WB, wb, 0)


def _deg_call(dstw):
    return pl.kernel(
        _deg_body,
        out_type=jax.ShapeDtypeStruct((2, NP, 16), _f32),
        mesh=_sc_mesh(),
        scratch_types=[
            pltpu.VMEM_SHARED((NP, 16), _f32),
            pltpu.VMEM((CPW1, CHUNK), jnp.int32),
            pltpu.VMEM((CHUNK, 16), _f32),
        ],
        compiler_params=_SC_PARAMS,
    )(dstw)


def _prop_body(y_hbm, srcw, dstw, out, acc, srcv, dstv, rows, sem):
    cid = lax.axis_index("c")
    sid = lax.axis_index("s")
    wid = sid * 2 + cid
    _fill(rows, CHUNK, H, 0.0)

    def z(k, c):
        rr = sid * RPT + k * CHUNK
        pltpu.sync_copy(rows, acc.at[pl.ds(rr, CHUNK)])
        return c

    lax.fori_loop(0, WB, z, 0)
    pltpu.sync_copy(srcw.at[wid], srcv)
    pltpu.sync_copy(dstw.at[wid], dstv)
    ncha = jnp.where(cid == 0, CPW0, CPW1)
    plsc.subcore_barrier()

    def chunk(j, c):
        pltpu.async_copy(y_hbm.at[srcv.at[j]], rows, sem).wait()
        pltpu.sync_copy(rows, acc.at[dstv.at[j]], add=True)
        return c

    lax.fori_loop(0, ncha, chunk, 0)
    plsc.subcore_barrier()

    def wb(k, c):
        rr = sid * RPT + k * CHUNK
        pltpu.sync_copy(acc.at[pl.ds(rr, CHUNK)], out.at[cid, pl.ds(rr, CHUNK)])
        return c

    lax.fori_loop(0, WB, wb, 0)


def _prop_call(y, srcw, dstw):
    return pl.kernel(
        _prop_body,
        out_type=jax.ShapeDtypeStruct((2, NP, H), _f32),
        mesh=_sc_mesh(),
        scratch_types=[
            pltpu.VMEM_SHARED((NP, H), _f32),
            pltpu.VMEM((CPW1, CHUNK), jnp.int32),
            pltpu.VMEM((CPW1, CHUNK), jnp.int32),
            pltpu.VMEM((CHUNK, H), _f32),
            pltpu.SemaphoreType.DMA,
        ],
        compiler_params=_SC_PARAMS,
    )(y, srcw, dstw)


def _tc_xw_body(x_ref, w1_ref, xw_ref):
    xw_ref[...] = jnp.dot(x_ref[...], w1_ref[...], preferred_element_type=_f32)


def _tc_xw_call(x_p, W1):
    return pl.pallas_call(
        _tc_xw_body,
        out_shape=jax.ShapeDtypeStruct((NP, H), _f32),
    )(x_p, W1)


def _tc_scale_body(xw_ref, dd_ref, y1_ref, dinv_ref):
    deg = dd_ref[0][:, 0:1] + dd_ref[1][:, 0:1] + 1.0
    dinvb = jnp.broadcast_to(lax.rsqrt(deg), (NP, H))
    y1_ref[...] = dinvb * xw_ref[...]
    dinv_ref[...] = dinvb


def _tc_scale_call(xw, dd):
    return pl.pallas_call(
        _tc_scale_body,
        out_shape=[
            jax.ShapeDtypeStruct((NP, H), _f32),
            jax.ShapeDtypeStruct((NP, H), _f32),
        ],
    )(xw, dd)


def _tc_c_body(p_ref, y1_ref, dinv_ref, b1_ref, w2_ref, y2_ref):
    dinvb = dinv_ref[...]
    h1 = jnp.maximum(dinvb * (p_ref[0] + p_ref[1] + y1_ref[...]) + b1_ref[...], 0.0)
    y2_ref[...] = dinvb * jnp.dot(h1, w2_ref[...], preferred_element_type=_f32)


def _tc_c_call(p, y1, dinvb, b1, W2):
    return pl.pallas_call(
        _tc_c_body,
        out_shape=jax.ShapeDtypeStruct((NP, H), _f32),
    )(p, y1, dinvb, b1, W2)


def _tc_d_body(q_ref, y2_ref, dinv_ref, b2_ref, batch_ref, fw1_ref, fb1_ref,
               fw2_ref, fb2_ref, out_ref):
    h2 = jnp.maximum(
        dinv_ref[...] * (q_ref[0] + q_ref[1] + y2_ref[...]) + b2_ref[...], 0.0)
    iot = lax.broadcasted_iota(jnp.int32, (NP, NG), 1)
    oh = (batch_ref[...] == iot).astype(_f32)
    ps = lax.dot_general(oh, h2, (((0,), (0,)), ((), ())),
                         preferred_element_type=_f32)
    ones = jnp.ones((NP, 1), _f32)
    cnt = lax.dot_general(oh, ones, (((0,), (0,)), ((), ())),
                          preferred_element_type=_f32)
    pooled = ps / jnp.maximum(cnt, 1.0)
    t = jnp.maximum(
        jnp.dot(pooled, fw1_ref[...], preferred_element_type=_f32)
        + fb1_ref[...], 0.0)
    logits = jnp.dot(t, fw2_ref[...], preferred_element_type=_f32) + fb2_ref[...]
    m = jnp.max(logits, axis=1, keepdims=True)
    lse = jnp.log(jnp.sum(jnp.exp(logits - m), axis=1, keepdims=True)) + m
    out_ref[...] = logits - lse


def _tc_d_call(q, y2, dinvb, b2, batch_p, fW1, fb1, fW2, fb2):
    return pl.pallas_call(
        _tc_d_body,
        out_shape=jax.ShapeDtypeStruct((NG, NCLS), _f32),
    )(q, y2, dinvb, b2, batch_p, fW1, fb1, fW2, fb2)


@jax.jit
def kernel(x, edge_index, batch, W1, b1, W2, b2, fW1, fb1, fW2, fb2):
    src = edge_index[0]
    dst = edge_index[1]

    def edge_layout(e, pad_val):
        e_p = jnp.concatenate([e, jnp.full((EP - E,), pad_val, jnp.int32)])
        n0 = 16 * CPW0 * CHUNK
        c0 = e_p[:n0].reshape(16, CPW0, CHUNK)
        c0 = jnp.pad(c0, ((0, 0), (0, CPW1 - CPW0), (0, 0)))
        c1 = e_p[n0:].reshape(16, CPW1, CHUNK)
        return jnp.stack([c0, c1], axis=1).reshape(NW, CPW1, CHUNK)

    src_p = edge_layout(src, 0)
    dst_p = edge_layout(dst, N)
    x_p = jnp.pad(x, ((0, NP - N), (0, 0)))
    batch_p = jnp.pad(batch, (0, NP - N), constant_values=NG).reshape(NP, 1)

    xw = _tc_xw_call(x_p, W1)
    dd = _deg_call(dst_p)
    y1, dinvb = _tc_scale_call(xw, dd)
    p = _prop_call(y1, src_p, dst_p)
    y2 = _tc_c_call(p, y1, dinvb, b1.reshape(1, H), W2)
    q = _prop_call(y2, src_p, dst_p)
    return _tc_d_call(q, y2, dinvb, b2.reshape(1, H), batch_p,
                      fW1, fb1.reshape(1, FC_H), fW2, fb2.reshape(1, NCLS))


# speedup vs baseline: 1.1250x; 1.1250x over previous
import jax
import jax.numpy as jnp
from jax import lax
from jax.experimental import pallas as pl
from jax.experimental.pallas import tpu as pltpu
from jax.experimental.pallas import tpu_sc as plsc

N = 10000
NP = 10240
E = 320000
D_IN = 128
H = 64
FC_H = 32
NCLS = 32
NG = 128

NW = 32
CHUNK = 128
CPW = 79
EP = NW * CPW * CHUNK
CPW0 = 94
CPW1 = 2 * CPW - CPW0
CPWMX = max(CPW0, CPW1)
RPT = NP // 16
WB = RPT // CHUNK

_f32 = jnp.float32


def _sc_mesh():
    return plsc.VectorSubcoreMesh(core_axis_name="c", subcore_axis_name="s")


_SC_PARAMS = pltpu.CompilerParams(use_tc_tiling_on_sc=False)


def _fill(buf, rows, cols, value):
    vals = jnp.full((16,), value, _f32)
    nc = cols // 16

    def st(k, c):
        buf[k // nc, pl.ds((k % nc) * 16, 16)] = vals
        return c

    lax.fori_loop(0, rows * nc, st, 0)


def _deg_body(dstw, out, accd, dstv, buf):
    cid = lax.axis_index("c")
    sid = lax.axis_index("s")
    wid = sid * 2 + cid
    _fill(buf, CHUNK, 16, 0.0)

    def z(k, c):
        rr = sid * RPT + k * CHUNK
        pltpu.sync_copy(buf, accd.at[pl.ds(rr, CHUNK)])
        return c

    lax.fori_loop(0, WB, z, 0)
    _fill(buf, CHUNK, 16, 1.0)
    pltpu.sync_copy(dstw.at[wid], dstv)
    ncha = jnp.where(cid == 0, CPW0, CPW1)
    plsc.subcore_barrier()

    def chunk(j, c):
        pltpu.sync_copy(buf, accd.at[dstv.at[j]], add=True)
        return c

    lax.fori_loop(0, ncha, chunk, 0)
    plsc.subcore_barrier()

    def wb(k, c):
        rr = sid * RPT + k * CHUNK
        pltpu.sync_copy(accd.at[pl.ds(rr, CHUNK)], out.at[cid, pl.ds(rr, CHUNK)])
        return c

    lax.fori_loop(0, WB, wb, 0)


def _deg_call(dstw):
    return pl.kernel(
        _deg_body,
        out_type=jax.ShapeDtypeStruct((2, NP, 16), _f32),
        mesh=_sc_mesh(),
        scratch_types=[
            pltpu.VMEM_SHARED((NP, 16), _f32),
            pltpu.VMEM((CPWMX, CHUNK), jnp.int32),
            pltpu.VMEM((CHUNK, 16), _f32),
        ],
        compiler_params=_SC_PARAMS,
    )(dstw)


def _prop_body(y_hbm, srcw, dstw, out, acc, srcv, dstv, rows, sem):
    cid = lax.axis_index("c")
    sid = lax.axis_index("s")
    wid = sid * 2 + cid
    _fill(rows, CHUNK, H, 0.0)

    def z(k, c):
        rr = sid * RPT + k * CHUNK
        pltpu.sync_copy(rows, acc.at[pl.ds(rr, CHUNK)])
        return c

    lax.fori_loop(0, WB, z, 0)
    pltpu.sync_copy(srcw.at[wid], srcv)
    pltpu.sync_copy(dstw.at[wid], dstv)
    ncha = jnp.where(cid == 0, CPW0, CPW1)
    plsc.subcore_barrier()

    def chunk(j, c):
        pltpu.async_copy(y_hbm.at[srcv.at[j]], rows, sem).wait()
        pltpu.sync_copy(rows, acc.at[dstv.at[j]], add=True)
        return c

    lax.fori_loop(0, ncha, chunk, 0)
    plsc.subcore_barrier()

    def wb(k, c):
        rr = sid * RPT + k * CHUNK
        pltpu.sync_copy(acc.at[pl.ds(rr, CHUNK)], out.at[cid, pl.ds(rr, CHUNK)])
        return c

    lax.fori_loop(0, WB, wb, 0)


def _prop_call(y, srcw, dstw):
    return pl.kernel(
        _prop_body,
        out_type=jax.ShapeDtypeStruct((2, NP, H), _f32),
        mesh=_sc_mesh(),
        scratch_types=[
            pltpu.VMEM_SHARED((NP, H), _f32),
            pltpu.VMEM((CPWMX, CHUNK), jnp.int32),
            pltpu.VMEM((CPWMX, CHUNK), jnp.int32),
            pltpu.VMEM((CHUNK, H), _f32),
            pltpu.SemaphoreType.DMA,
        ],
        compiler_params=_SC_PARAMS,
    )(y, srcw, dstw)


def _tc_xw_body(x_ref, w1_ref, xw_ref):
    xw_ref[...] = jnp.dot(x_ref[...], w1_ref[...], preferred_element_type=_f32)


def _tc_xw_call(x_p, W1):
    return pl.pallas_call(
        _tc_xw_body,
        out_shape=jax.ShapeDtypeStruct((NP, H), _f32),
    )(x_p, W1)


def _tc_scale_body(xw_ref, dd_ref, y1_ref, dinv_ref):
    deg = dd_ref[0][:, 0:1] + dd_ref[1][:, 0:1] + 1.0
    dinvb = jnp.broadcast_to(lax.rsqrt(deg), (NP, H))
    y1_ref[...] = dinvb * xw_ref[...]
    dinv_ref[...] = dinvb


def _tc_scale_call(xw, dd):
    return pl.pallas_call(
        _tc_scale_body,
        out_shape=[
            jax.ShapeDtypeStruct((NP, H), _f32),
            jax.ShapeDtypeStruct((NP, H), _f32),
        ],
    )(xw, dd)


def _tc_c_body(p_ref, y1_ref, dinv_ref, b1_ref, w2_ref, y2_ref):
    dinvb = dinv_ref[...]
    h1 = jnp.maximum(dinvb * (p_ref[0] + p_ref[1] + y1_ref[...]) + b1_ref[...], 0.0)
    y2_ref[...] = dinvb * jnp.dot(h1, w2_ref[...], preferred_element_type=_f32)


def _tc_c_call(p, y1, dinvb, b1, W2):
    return pl.pallas_call(
        _tc_c_body,
        out_shape=jax.ShapeDtypeStruct((NP, H), _f32),
    )(p, y1, dinvb, b1, W2)


def _tc_d_body(q_ref, y2_ref, dinv_ref, b2_ref, batch_ref, fw1_ref, fb1_ref,
               fw2_ref, fb2_ref, out_ref):
    h2 = jnp.maximum(
        dinv_ref[...] * (q_ref[0] + q_ref[1] + y2_ref[...]) + b2_ref[...], 0.0)
    iot = lax.broadcasted_iota(jnp.int32, (NP, NG), 1)
    oh = (batch_ref[...] == iot).astype(_f32)
    ps = lax.dot_general(oh, h2, (((0,), (0,)), ((), ())),
                         preferred_element_type=_f32)
    ones = jnp.ones((NP, 1), _f32)
    cnt = lax.dot_general(oh, ones, (((0,), (0,)), ((), ())),
                          preferred_element_type=_f32)
    pooled = ps / jnp.maximum(cnt, 1.0)
    t = jnp.maximum(
        jnp.dot(pooled, fw1_ref[...], preferred_element_type=_f32)
        + fb1_ref[...], 0.0)
    logits = jnp.dot(t, fw2_ref[...], preferred_element_type=_f32) + fb2_ref[...]
    m = jnp.max(logits, axis=1, keepdims=True)
    lse = jnp.log(jnp.sum(jnp.exp(logits - m), axis=1, keepdims=True)) + m
    out_ref[...] = logits - lse


def _tc_d_call(q, y2, dinvb, b2, batch_p, fW1, fb1, fW2, fb2):
    return pl.pallas_call(
        _tc_d_body,
        out_shape=jax.ShapeDtypeStruct((NG, NCLS), _f32),
    )(q, y2, dinvb, b2, batch_p, fW1, fb1, fW2, fb2)


@jax.jit
def kernel(x, edge_index, batch, W1, b1, W2, b2, fW1, fb1, fW2, fb2):
    src = edge_index[0]
    dst = edge_index[1]

    def edge_layout(e, pad_val):
        e_p = jnp.concatenate([e, jnp.full((EP - E,), pad_val, jnp.int32)])
        n0 = 16 * CPW0 * CHUNK
        c0 = e_p[:n0].reshape(16, CPW0, CHUNK)
        c0 = jnp.pad(c0, ((0, 0), (0, CPWMX - CPW0), (0, 0)))
        c1 = e_p[n0:].reshape(16, CPW1, CHUNK)
        c1 = jnp.pad(c1, ((0, 0), (0, CPWMX - CPW1), (0, 0)))
        return jnp.stack([c0, c1], axis=1).reshape(NW, CPWMX, CHUNK)

    src_p = edge_layout(src, 0)
    dst_p = edge_layout(dst, N)
    x_p = jnp.pad(x, ((0, NP - N), (0, 0)))
    batch_p = jnp.pad(batch, (0, NP - N), constant_values=NG).reshape(NP, 1)

    xw = _tc_xw_call(x_p, W1)
    dd = _deg_call(dst_p)
    y1, dinvb = _tc_scale_call(xw, dd)
    p = _prop_call(y1, src_p, dst_p)
    y2 = _tc_c_call(p, y1, dinvb, b1.reshape(1, H), W2)
    q = _prop_call(y2, src_p, dst_p)
    return _tc_d_call(q, y2, dinvb, b2.reshape(1, H), batch_p,
                      fW1, fb1.reshape(1, FC_H), fW2, fb2.reshape(1, NCLS))


# speedup vs baseline: 1.1963x; 1.0634x over previous
import jax
import jax.numpy as jnp
from jax import lax
from jax.experimental import pallas as pl
from jax.experimental.pallas import tpu as pltpu
from jax.experimental.pallas import tpu_sc as plsc

N = 10000
NP = 10240
E = 320000
D_IN = 128
H = 64
FC_H = 32
NCLS = 32
NG = 128

NW = 32
CHUNK = 128
CPW = 79
EP = NW * CPW * CHUNK
CPW0 = 90
CPW1 = 2 * CPW - CPW0
CPWMX = max(CPW0, CPW1)
RPT = NP // 16
WB = RPT // CHUNK

_f32 = jnp.float32


def _sc_mesh():
    return plsc.VectorSubcoreMesh(core_axis_name="c", subcore_axis_name="s")


_SC_PARAMS = pltpu.CompilerParams(use_tc_tiling_on_sc=False)


def _fill(buf, rows, cols, value):
    vals = jnp.full((16,), value, _f32)
    nc = cols // 16

    def st(k, c):
        buf[k // nc, pl.ds((k % nc) * 16, 16)] = vals
        return c

    lax.fori_loop(0, rows * nc, st, 0)


def _deg_body(dstw, out, accd, dstv, buf):
    cid = lax.axis_index("c")
    sid = lax.axis_index("s")
    wid = sid * 2 + cid
    _fill(buf, CHUNK, 16, 0.0)

    def z(k, c):
        rr = sid * RPT + k * CHUNK
        pltpu.sync_copy(buf, accd.at[pl.ds(rr, CHUNK)])
        return c

    lax.fori_loop(0, WB, z, 0)
    _fill(buf, CHUNK, 16, 1.0)
    pltpu.sync_copy(dstw.at[wid], dstv)
    ncha = jnp.where(cid == 0, CPW0, CPW1)
    plsc.subcore_barrier()

    def chunk(j, c):
        pltpu.sync_copy(buf, accd.at[dstv.at[j]], add=True)
        return c

    lax.fori_loop(0, ncha, chunk, 0)
    plsc.subcore_barrier()

    def wb(k, c):
        rr = sid * RPT + k * CHUNK
        pltpu.sync_copy(accd.at[pl.ds(rr, CHUNK)],
                        out.at[pl.ds(rr, CHUNK), pl.ds(cid * 16, 16)])
        return c

    lax.fori_loop(0, WB, wb, 0)


def _deg_call(dstw):
    return pl.kernel(
        _deg_body,
        out_type=jax.ShapeDtypeStruct((NP, 32), _f32),
        mesh=_sc_mesh(),
        scratch_types=[
            pltpu.VMEM_SHARED((NP, 16), _f32),
            pltpu.VMEM((CPWMX, CHUNK), jnp.int32),
            pltpu.VMEM((CHUNK, 16), _f32),
        ],
        compiler_params=_SC_PARAMS,
    )(dstw)


def _prop_body(y_hbm, srcw, dstw, out, acc, srcv, dstv, rows, sem):
    cid = lax.axis_index("c")
    sid = lax.axis_index("s")
    wid = sid * 2 + cid
    _fill(rows, CHUNK, H, 0.0)

    def z(k, c):
        rr = sid * RPT + k * CHUNK
        pltpu.sync_copy(rows, acc.at[pl.ds(rr, CHUNK)])
        return c

    lax.fori_loop(0, WB, z, 0)
    pltpu.sync_copy(srcw.at[wid], srcv)
    pltpu.sync_copy(dstw.at[wid], dstv)
    ncha = jnp.where(cid == 0, CPW0, CPW1)
    plsc.subcore_barrier()

    def chunk(j, c):
        pltpu.async_copy(y_hbm.at[srcv.at[j]], rows, sem).wait()
        pltpu.sync_copy(rows, acc.at[dstv.at[j]], add=True)
        return c

    lax.fori_loop(0, ncha, chunk, 0)
    plsc.subcore_barrier()

    def wb(k, c):
        rr = sid * RPT + k * CHUNK
        pltpu.sync_copy(acc.at[pl.ds(rr, CHUNK)],
                        out.at[pl.ds(rr, CHUNK), pl.ds(cid * H, H)])
        return c

    lax.fori_loop(0, WB, wb, 0)


def _prop_call(y, srcw, dstw):
    return pl.kernel(
        _prop_body,
        out_type=jax.ShapeDtypeStruct((NP, 2 * H), _f32),
        mesh=_sc_mesh(),
        scratch_types=[
            pltpu.VMEM_SHARED((NP, H), _f32),
            pltpu.VMEM((CPWMX, CHUNK), jnp.int32),
            pltpu.VMEM((CPWMX, CHUNK), jnp.int32),
            pltpu.VMEM((CHUNK, H), _f32),
            pltpu.SemaphoreType.DMA,
        ],
        compiler_params=_SC_PARAMS,
    )(y, srcw, dstw)


def _tc_xw_body(x_ref, w1_ref, xw_ref):
    xw_ref[...] = jnp.dot(x_ref[...], w1_ref[...], preferred_element_type=_f32)


def _tc_xw_call(x_p, W1):
    return pl.pallas_call(
        _tc_xw_body,
        out_shape=jax.ShapeDtypeStruct((NP, H), _f32),
    )(x_p, W1)


def _tc_scale_body(xw_ref, dd_ref, y1_ref, dinv_ref):
    deg = dd_ref[:, 0:1] + dd_ref[:, 16:17] + 1.0
    dinvb = jnp.broadcast_to(lax.rsqrt(deg), (NP, H))
    y1_ref[...] = dinvb * xw_ref[...]
    dinv_ref[...] = dinvb


def _tc_scale_call(xw, dd):
    return pl.pallas_call(
        _tc_scale_body,
        out_shape=[
            jax.ShapeDtypeStruct((NP, H), _f32),
            jax.ShapeDtypeStruct((NP, H), _f32),
        ],
    )(xw, dd)


def _tc_c_body(p_ref, y1_ref, dinv_ref, b1_ref, w2_ref, y2_ref):
    dinvb = dinv_ref[...]
    h1 = jnp.maximum(
        dinvb * (p_ref[:, :H] + p_ref[:, H:] + y1_ref[...]) + b1_ref[...], 0.0)
    y2_ref[...] = dinvb * jnp.dot(h1, w2_ref[...], preferred_element_type=_f32)


def _tc_c_call(p, y1, dinvb, b1, W2):
    return pl.pallas_call(
        _tc_c_body,
        out_shape=jax.ShapeDtypeStruct((NP, H), _f32),
    )(p, y1, dinvb, b1, W2)


def _tc_d_body(q_ref, y2_ref, dinv_ref, b2_ref, batch_ref, fw1_ref, fb1_ref,
               fw2_ref, fb2_ref, out_ref):
    h2 = jnp.maximum(
        dinv_ref[...] * (q_ref[:, :H] + q_ref[:, H:] + y2_ref[...])
        + b2_ref[...], 0.0)
    iot = lax.broadcasted_iota(jnp.int32, (NP, NG), 1)
    oh = (batch_ref[...] == iot).astype(_f32)
    ps = lax.dot_general(oh, h2, (((0,), (0,)), ((), ())),
                         preferred_element_type=_f32)
    ones = jnp.ones((NP, 1), _f32)
    cnt = lax.dot_general(oh, ones, (((0,), (0,)), ((), ())),
                          preferred_element_type=_f32)
    pooled = ps / jnp.maximum(cnt, 1.0)
    t = jnp.maximum(
        jnp.dot(pooled, fw1_ref[...], preferred_element_type=_f32)
        + fb1_ref[...], 0.0)
    logits = jnp.dot(t, fw2_ref[...], preferred_element_type=_f32) + fb2_ref[...]
    m = jnp.max(logits, axis=1, keepdims=True)
    lse = jnp.log(jnp.sum(jnp.exp(logits - m), axis=1, keepdims=True)) + m
    out_ref[...] = logits - lse


def _tc_d_call(q, y2, dinvb, b2, batch_p, fW1, fb1, fW2, fb2):
    return pl.pallas_call(
        _tc_d_body,
        out_shape=jax.ShapeDtypeStruct((NG, NCLS), _f32),
    )(q, y2, dinvb, b2, batch_p, fW1, fb1, fW2, fb2)


@jax.jit
def kernel(x, edge_index, batch, W1, b1, W2, b2, fW1, fb1, fW2, fb2):
    src = edge_index[0]
    dst = edge_index[1]

    def edge_layout(e, pad_val):
        e_p = jnp.concatenate([e, jnp.full((EP - E,), pad_val, jnp.int32)])
        n0 = 16 * CPW0 * CHUNK
        c0 = e_p[:n0].reshape(16, CPW0, CHUNK)
        c0 = jnp.pad(c0, ((0, 0), (0, CPWMX - CPW0), (0, 0)))
        c1 = e_p[n0:].reshape(16, CPW1, CHUNK)
        c1 = jnp.pad(c1, ((0, 0), (0, CPWMX - CPW1), (0, 0)))
        return jnp.stack([c0, c1], axis=1).reshape(NW, CPWMX, CHUNK)

    src_p = edge_layout(src, 0)
    dst_p = edge_layout(dst, N)
    x_p = jnp.pad(x, ((0, NP - N), (0, 0)))
    batch_p = jnp.pad(batch, (0, NP - N), constant_values=NG).reshape(NP, 1)

    xw = _tc_xw_call(x_p, W1)
    dd = _deg_call(dst_p)
    y1, dinvb = _tc_scale_call(xw, dd)
    p = _prop_call(y1, src_p, dst_p)
    y2 = _tc_c_call(p, y1, dinvb, b1.reshape(1, H), W2)
    q = _prop_call(y2, src_p, dst_p)
    return _tc_d_call(q, y2, dinvb, b2.reshape(1, H), batch_p,
                      fW1, fb1.reshape(1, FC_H), fW2, fb2.reshape(1, NCLS))
